# Initial kernel scaffold; baseline (speedup 1.0000x reference)
import functools

import jax
import jax.numpy as jnp
from jax import lax
from jax.experimental import pallas as pl
from jax.experimental.pallas import tpu as pltpu
from jax.experimental.pallas import tpu_sc as plsc

_NC = 2
_NS = 16
_NW = _NC * _NS
_LW = 128

_N = 10000
_E = 320000
_D = 128

_ROWS = ((_E + _NW * _LW - 1) // (_NW * _LW)) * _NW
_EPAD = _ROWS * _LW
_RPT = _ROWS // _NW

_NPAD = ((_N + _NW) + 15) // 16 * 16
_ZROWS = _NPAD // _NS
_OROWS = _N // _NS

_mesh = plsc.VectorSubcoreMesh(
    core_axis_name="c", subcore_axis_name="s", num_cores=_NC, num_subcores=_NS
)


@functools.partial(
    pl.kernel,
    out_type=jax.ShapeDtypeStruct((_NC, _N, 16), jnp.float32),
    mesh=_mesh,
    scratch_types=[
        pltpu.VMEM((_RPT, 1, _LW), jnp.int32),
        pltpu.VMEM((_LW, 16), jnp.float32),
        pltpu.VMEM_SHARED((_NPAD, 16), jnp.float32),
    ],
)
def _sc_degree(d_hbm, ones_hbm, zeros_hbm, out_hbm, dv, ones_v, acc):
    cid = lax.axis_index("c")
    sid = lax.axis_index("s")
    wid = cid * _NS + sid
    pltpu.sync_copy(zeros_hbm, acc.at[pl.ds(sid * _ZROWS, _ZROWS)])
    pltpu.sync_copy(d_hbm.at[pl.ds(wid * _RPT, _RPT)], dv)
    pltpu.sync_copy(ones_hbm, ones_v)
    plsc.subcore_barrier()

    @pl.loop(0, _RPT)
    def _(r):
        pltpu.sync_copy(ones_v, acc.at[dv.at[r]], add=True)

    plsc.subcore_barrier()
    pltpu.sync_copy(
        acc.at[pl.ds(sid * _OROWS, _OROWS)],
        out_hbm.at[cid].at[pl.ds(sid * _OROWS, _OROWS)],
    )


@functools.partial(
    pl.kernel,
    out_type=jax.ShapeDtypeStruct((_NC, _N, _D), jnp.float32),
    mesh=_mesh,
    scratch_types=[
        pltpu.VMEM((_RPT, _LW), jnp.int32),
        pltpu.VMEM((_RPT, 1, _LW), jnp.int32),
        pltpu.VMEM((_LW, _D), jnp.float32),
        pltpu.VMEM((_LW, _D), jnp.float32),
        pltpu.VMEM_SHARED((_NPAD, _D), jnp.float32),
        pltpu.SemaphoreType.DMA,
        pltpu.SemaphoreType.DMA,
    ],
)
def _sc_scatter(z_hbm, s_hbm, d_hbm, zeros_hbm, out_hbm,
                sv, dv, rb0, rb1, acc, sem0, sem1):
    cid = lax.axis_index("c")
    sid = lax.axis_index("s")
    wid = cid * _NS + sid
    pltpu.sync_copy(zeros_hbm, acc.at[pl.ds(sid * _ZROWS, _ZROWS)])
    pltpu.sync_copy(s_hbm.at[pl.ds(wid * _RPT, _RPT)], sv)
    pltpu.sync_copy(d_hbm.at[pl.ds(wid * _RPT, _RPT)], dv)
    plsc.subcore_barrier()

    pltpu.async_copy(z_hbm.at[sv.at[0]], rb0, sem0)

    @pl.loop(0, _RPT, step=2)
    def _(r):
        pltpu.async_copy(z_hbm.at[sv.at[r + 1]], rb1, sem1)
        pltpu.make_async_copy(z_hbm.at[sv.at[r]], rb0, sem0).wait()
        pltpu.sync_copy(rb0, acc.at[dv.at[r]], add=True)

        @pl.when(r + 2 < _RPT)
        def _():
            pltpu.async_copy(z_hbm.at[sv.at[r + 2]], rb0, sem0)

        pltpu.make_async_copy(z_hbm.at[sv.at[r + 1]], rb1, sem1).wait()
        pltpu.sync_copy(rb1, acc.at[dv.at[r + 1]], add=True)

    plsc.subcore_barrier()
    pltpu.sync_copy(
        acc.at[pl.ds(sid * _OROWS, _OROWS)],
        out_hbm.at[cid].at[pl.ds(sid * _OROWS, _OROWS)],
    )


_BR = 500


def _dot(a, b):
    return lax.dot_general(
        a, b, (((1,), (0,)), ((), ())),
        precision=lax.Precision.HIGHEST,
        preferred_element_type=jnp.float32,
    )


def _tc_first_body(x_ref, c0_ref, c1_ref, w_ref, z_ref, dinv_ref):
    deg = c0_ref[:, 0:1] + c1_ref[:, 0:1] + 1.0
    dinv = lax.rsqrt(deg)
    z_ref[...] = _dot(x_ref[...], w_ref[...]) * dinv
    dinv_ref[...] = dinv


def _tc_first(x, c0, c1, w0):
    return pl.pallas_call(
        _tc_first_body,
        grid=(_N // _BR,),
        in_specs=[
            pl.BlockSpec((_BR, _D), lambda i: (i, 0)),
            pl.BlockSpec((_BR, 16), lambda i: (i, 0)),
            pl.BlockSpec((_BR, 16), lambda i: (i, 0)),
            pl.BlockSpec((_D, _D), lambda i: (0, 0)),
        ],
        out_specs=[
            pl.BlockSpec((_BR, _D), lambda i: (i, 0)),
            pl.BlockSpec((_BR, 1), lambda i: (i, 0)),
        ],
        out_shape=[
            jax.ShapeDtypeStruct((_N, _D), jnp.float32),
            jax.ShapeDtypeStruct((_N, 1), jnp.float32),
        ],
    )(x, c0, c1, w0)


def _tc_mid_body(ma_ref, mb_ref, z_ref, dinv_ref, b_ref, w_ref, o_ref):
    dinv = dinv_ref[...]
    t = (ma_ref[...] + mb_ref[...] + z_ref[...]) * dinv + b_ref[...]
    h = jnp.maximum(t, 0.0)
    o_ref[...] = _dot(h, w_ref[...]) * dinv


def _tc_mid(ma, mb, z, dinv, b, w):
    return pl.pallas_call(
        _tc_mid_body,
        grid=(_N // _BR,),
        in_specs=[
            pl.BlockSpec((_BR, _D), lambda i: (i, 0)),
            pl.BlockSpec((_BR, _D), lambda i: (i, 0)),
            pl.BlockSpec((_BR, _D), lambda i: (i, 0)),
            pl.BlockSpec((_BR, 1), lambda i: (i, 0)),
            pl.BlockSpec((1, _D), lambda i: (0, 0)),
            pl.BlockSpec((_D, _D), lambda i: (0, 0)),
        ],
        out_specs=pl.BlockSpec((_BR, _D), lambda i: (i, 0)),
        out_shape=jax.ShapeDtypeStruct((_N, _D), jnp.float32),
    )(ma, mb, z, dinv, b, w)


def _tc_last_body(ma_ref, mb_ref, z_ref, dinv_ref, b_ref, w_ref, bc_ref, o_ref):
    t = (ma_ref[...] + mb_ref[...] + z_ref[...]) * dinv_ref[...] + b_ref[...]
    h = jnp.maximum(t, 0.0)
    o_ref[...] = _dot(h, w_ref[...]) + bc_ref[...]


def _tc_last(ma, mb, z, dinv, b, wc, bc):
    dout = wc.shape[1]
    return pl.pallas_call(
        _tc_last_body,
        grid=(_N // _BR,),
        in_specs=[
            pl.BlockSpec((_BR, _D), lambda i: (i, 0)),
            pl.BlockSpec((_BR, _D), lambda i: (i, 0)),
            pl.BlockSpec((_BR, _D), lambda i: (i, 0)),
            pl.BlockSpec((_BR, 1), lambda i: (i, 0)),
            pl.BlockSpec((1, _D), lambda i: (0, 0)),
            pl.BlockSpec((_D, dout), lambda i: (0, 0)),
            pl.BlockSpec((1, dout), lambda i: (0, 0)),
        ],
        out_specs=pl.BlockSpec((_BR, dout), lambda i: (i, 0)),
        out_shape=jax.ShapeDtypeStruct((_N, dout), jnp.float32),
    )(ma, mb, z, dinv, b, wc, bc)


def kernel(x, edge_index, W0, b0, W1, b1, W2, b2, Wc, bc):
    s = edge_index[0].astype(jnp.int32)
    d = edge_index[1].astype(jnp.int32)
    npad = _EPAD - _E
    s_rows = jnp.concatenate([s, jnp.zeros((npad,), jnp.int32)]).reshape(
        _ROWS, _LW)
    d_rows = jnp.concatenate([d, jnp.full((npad,), _N, jnp.int32)]).reshape(
        _ROWS, _LW)
    d_rows3 = d_rows.reshape(_ROWS, 1, _LW)

    ones16 = jnp.ones((_LW, 16), jnp.float32)
    zeros16 = jnp.zeros((_ZROWS, 16), jnp.float32)
    zeros128 = jnp.zeros((_ZROWS, _D), jnp.float32)

    cnt = _sc_degree(d_rows3, ones16, zeros16)
    z0, dinv = _tc_first(x, cnt[0], cnt[1], W0)

    m0 = _sc_scatter(z0, s_rows, d_rows3, zeros128)
    z1 = _tc_mid(m0[0], m0[1], z0, dinv, b0.reshape(1, _D), W1)

    m1 = _sc_scatter(z1, s_rows, d_rows3, zeros128)
    z2 = _tc_mid(m1[0], m1[1], z1, dinv, b1.reshape(1, _D), W2)

    m2 = _sc_scatter(z2, s_rows, d_rows3, zeros128)
    return _tc_last(m2[0], m2[1], z2, dinv, b2.reshape(1, _D), Wc, bc)


# speedup vs baseline: 19.9623x; 19.9623x over previous
import functools

import jax
import jax.numpy as jnp
from jax import lax
from jax.experimental import pallas as pl
from jax.experimental.pallas import tpu as pltpu
from jax.experimental.pallas import tpu_sc as plsc

_NC = 2
_NS = 16
_NW = _NC * _NS
_LW = 128

_N = 10000
_E = 320000
_D = 128
_DH = _D // 2

_RPD = -2 * (-_E // (_NW * _LW * 2))
_ROWS = _RPD * _NW
_EPAD = _ROWS * _LW
_RPS = _ROWS // _NS

_ZROWS = -8 * (-(_N + _NW) // (_NS * 8))
_NPAD = _ZROWS * _NS

_mesh = plsc.VectorSubcoreMesh(
    core_axis_name="c", subcore_axis_name="s", num_cores=_NC, num_subcores=_NS
)
_sc_params = pltpu.CompilerParams(use_tc_tiling_on_sc=False)


@functools.partial(
    pl.kernel,
    out_type=jax.ShapeDtypeStruct((_NC, _NPAD, 16), jnp.float32),
    mesh=_mesh,
    compiler_params=_sc_params,
    scratch_types=[
        pltpu.VMEM((_RPD, _LW), jnp.int32),
        pltpu.VMEM((_LW, 16), jnp.float32),
        pltpu.VMEM_SHARED((_NPAD, 16), jnp.float32),
    ],
)
def _sc_degree(d_hbm, ones_hbm, zeros_hbm, out_hbm, dv, ones_v, acc):
    cid = lax.axis_index("c")
    sid = lax.axis_index("s")
    wid = cid * _NS + sid
    pltpu.sync_copy(zeros_hbm, acc.at[pl.ds(sid * _ZROWS, _ZROWS)])
    pltpu.sync_copy(d_hbm.at[pl.ds(wid * _RPD, _RPD)], dv)
    pltpu.sync_copy(ones_hbm, ones_v)
    plsc.subcore_barrier()

    @pl.loop(0, _RPD)
    def _(r):
        pltpu.sync_copy(ones_v, acc.at[dv.at[r]], add=True)

    plsc.subcore_barrier()
    pltpu.sync_copy(
        acc.at[pl.ds(sid * _ZROWS, _ZROWS)],
        out_hbm.at[cid].at[pl.ds(sid * _ZROWS, _ZROWS)],
    )


@functools.partial(
    pl.kernel,
    out_type=jax.ShapeDtypeStruct((_NC, _NPAD, _DH), jnp.float32),
    mesh=_mesh,
    compiler_params=_sc_params,
    scratch_types=[
        pltpu.VMEM((_RPS, _LW), jnp.int32),
        pltpu.VMEM((_RPS, _LW), jnp.int32),
        pltpu.VMEM((_LW, _DH), jnp.float32),
        pltpu.VMEM((_LW, _DH), jnp.float32),
        pltpu.VMEM_SHARED((_NPAD, _DH), jnp.float32),
        pltpu.SemaphoreType.DMA,
        pltpu.SemaphoreType.DMA,
    ],
)
def _sc_scatter(z_hbm, s_hbm, d_hbm, zeros_hbm, out_hbm,
                sv, dv, rb0, rb1, acc, sem0, sem1):
    cid = lax.axis_index("c")
    sid = lax.axis_index("s")
    zc = z_hbm.at[cid]
    pltpu.sync_copy(zeros_hbm, acc.at[pl.ds(sid * _ZROWS, _ZROWS)])
    pltpu.sync_copy(s_hbm.at[pl.ds(sid * _RPS, _RPS)], sv)
    pltpu.sync_copy(d_hbm.at[pl.ds(sid * _RPS, _RPS)], dv)
    plsc.subcore_barrier()

    pltpu.async_copy(zc.at[sv.at[0]], rb0, sem0)

    @pl.loop(0, _RPS, step=2)
    def _(r):
        pltpu.async_copy(zc.at[sv.at[r + 1]], rb1, sem1)
        pltpu.make_async_copy(zc.at[sv.at[r]], rb0, sem0).wait()
        pltpu.sync_copy(rb0, acc.at[dv.at[r]], add=True)

        @pl.when(r + 2 < _RPS)
        def _():
            pltpu.async_copy(zc.at[sv.at[r + 2]], rb0, sem0)

        pltpu.make_async_copy(zc.at[sv.at[r + 1]], rb1, sem1).wait()
        pltpu.sync_copy(rb1, acc.at[dv.at[r + 1]], add=True)

    plsc.subcore_barrier()
    pltpu.sync_copy(
        acc.at[pl.ds(sid * _ZROWS, _ZROWS)],
        out_hbm.at[cid].at[pl.ds(sid * _ZROWS, _ZROWS)],
    )


_BR = 1000


def _dot(a, b):
    return lax.dot_general(
        a, b, (((1,), (0,)), ((), ())),
        precision=lax.Precision.HIGHEST,
        preferred_element_type=jnp.float32,
    )


def _split_store(z2_ref, z):
    z2_ref[0] = z[:, :_DH]
    z2_ref[1] = z[:, _DH:]


def _tc_first_body(x_ref, c0_ref, c1_ref, w_ref, z2_ref, dinv_ref):
    deg = c0_ref[:, 0:1] + c1_ref[:, 0:1] + 1.0
    dinv = lax.rsqrt(deg)
    _split_store(z2_ref, _dot(x_ref[...], w_ref[...]) * dinv)
    dinv_ref[...] = dinv


def _tc_first(x, c0, c1, w0):
    return pl.pallas_call(
        _tc_first_body,
        grid=(_N // _BR,),
        in_specs=[
            pl.BlockSpec((_BR, _D), lambda i: (i, 0)),
            pl.BlockSpec((_BR, 16), lambda i: (i, 0)),
            pl.BlockSpec((_BR, 16), lambda i: (i, 0)),
            pl.BlockSpec((_D, _D), lambda i: (0, 0)),
        ],
        out_specs=[
            pl.BlockSpec((_NC, _BR, _DH), lambda i: (0, i, 0)),
            pl.BlockSpec((_BR, 1), lambda i: (i, 0)),
        ],
        out_shape=[
            jax.ShapeDtypeStruct((_NC, _N, _DH), jnp.float32),
            jax.ShapeDtypeStruct((_N, 1), jnp.float32),
        ],
    )(x, c0, c1, w0)


def _relu_cat(m_ref, z_ref, dinv, b_ref):
    t = jnp.concatenate(
        [m_ref[0] + z_ref[0], m_ref[1] + z_ref[1]], axis=1
    ) * dinv + b_ref[...]
    return jnp.maximum(t, 0.0)


def _tc_mid_body(m_ref, z_ref, dinv_ref, b_ref, w_ref, o_ref):
    dinv = dinv_ref[...]
    h = _relu_cat(m_ref, z_ref, dinv, b_ref)
    _split_store(o_ref, _dot(h, w_ref[...]) * dinv)


def _tc_mid(m, z2, dinv, b, w):
    return pl.pallas_call(
        _tc_mid_body,
        grid=(_N // _BR,),
        in_specs=[
            pl.BlockSpec((_NC, _BR, _DH), lambda i: (0, i, 0)),
            pl.BlockSpec((_NC, _BR, _DH), lambda i: (0, i, 0)),
            pl.BlockSpec((_BR, 1), lambda i: (i, 0)),
            pl.BlockSpec((1, _D), lambda i: (0, 0)),
            pl.BlockSpec((_D, _D), lambda i: (0, 0)),
        ],
        out_specs=pl.BlockSpec((_NC, _BR, _DH), lambda i: (0, i, 0)),
        out_shape=jax.ShapeDtypeStruct((_NC, _N, _DH), jnp.float32),
    )(m, z2, dinv, b, w)


def _tc_last_body(m_ref, z_ref, dinv_ref, b_ref, w_ref, bc_ref, o_ref):
    h = _relu_cat(m_ref, z_ref, dinv_ref[...], b_ref)
    o_ref[...] = _dot(h, w_ref[...]) + bc_ref[...]


def _tc_last(m, z2, dinv, b, wc, bc):
    dout = wc.shape[1]
    return pl.pallas_call(
        _tc_last_body,
        grid=(_N // _BR,),
        in_specs=[
            pl.BlockSpec((_NC, _BR, _DH), lambda i: (0, i, 0)),
            pl.BlockSpec((_NC, _BR, _DH), lambda i: (0, i, 0)),
            pl.BlockSpec((_BR, 1), lambda i: (i, 0)),
            pl.BlockSpec((1, _D), lambda i: (0, 0)),
            pl.BlockSpec((_D, dout), lambda i: (0, 0)),
            pl.BlockSpec((1, dout), lambda i: (0, 0)),
        ],
        out_specs=pl.BlockSpec((_BR, dout), lambda i: (i, 0)),
        out_shape=jax.ShapeDtypeStruct((_N, dout), jnp.float32),
    )(m, z2, dinv, b, wc, bc.reshape(1, dout))


def kernel(x, edge_index, W0, b0, W1, b1, W2, b2, Wc, bc):
    s = edge_index[0].astype(jnp.int32)
    d = edge_index[1].astype(jnp.int32)
    npad = _EPAD - _E
    pad_src = jnp.arange(npad, dtype=jnp.int32) % _N
    pad_dst = _N + (jnp.arange(npad, dtype=jnp.int32) % _NW)
    s_rows = jnp.concatenate([s, pad_src]).reshape(_ROWS, _LW)
    d_rows = jnp.concatenate([d, pad_dst]).reshape(_ROWS, _LW)

    ones16 = jnp.ones((_LW, 16), jnp.float32)
    zeros16 = jnp.zeros((_ZROWS, 16), jnp.float32)
    zeros64 = jnp.zeros((_ZROWS, _DH), jnp.float32)

    def _sc(z2):
        return _sc_scatter(z2, s_rows, d_rows, zeros64)[:, :_N]

    cnt = _sc_degree(d_rows, ones16, zeros16)
    z0, dinv = _tc_first(x, cnt[0, :_N], cnt[1, :_N], W0)

    z1 = _tc_mid(_sc(z0), z0, dinv, b0.reshape(1, _D), W1)
    z2 = _tc_mid(_sc(z1), z1, dinv, b1.reshape(1, _D), W2)
    return _tc_last(_sc(z2), z2, dinv, b2.reshape(1, _D), Wc, bc)


# speedup vs baseline: 22.0532x; 1.1047x over previous
import functools

import jax
import jax.numpy as jnp
from jax import lax
from jax.experimental import pallas as pl
from jax.experimental.pallas import tpu as pltpu
from jax.experimental.pallas import tpu_sc as plsc

_NC = 2
_NS = 16
_NW = _NC * _NS
_LW = 128

_N = 10000
_E = 320000
_D = 128
_DH = _D // 2

_RPD = -2 * (-_E // (_NW * _LW * 2))
_ROWS = _RPD * _NW
_EPAD = _ROWS * _LW
_RPS = _ROWS // _NS

_ZROWS = -8 * (-(_N + _NW) // (_NS * 8))
_NPAD = _ZROWS * _NS

_mesh = plsc.VectorSubcoreMesh(
    core_axis_name="c", subcore_axis_name="s", num_cores=_NC, num_subcores=_NS
)
_sc_params = pltpu.CompilerParams(use_tc_tiling_on_sc=False)


@functools.partial(
    pl.kernel,
    out_type=jax.ShapeDtypeStruct((_NC, _NPAD, 16), jnp.float32),
    mesh=_mesh,
    compiler_params=_sc_params,
    scratch_types=[
        pltpu.VMEM((_RPD, _LW), jnp.int32),
        pltpu.VMEM((_LW, 16), jnp.float32),
        pltpu.VMEM_SHARED((_NPAD, 16), jnp.float32),
    ],
)
def _sc_degree(d_hbm, ones_hbm, zeros_hbm, out_hbm, dv, ones_v, acc):
    cid = lax.axis_index("c")
    sid = lax.axis_index("s")
    wid = cid * _NS + sid
    pltpu.sync_copy(zeros_hbm, acc.at[pl.ds(sid * _ZROWS, _ZROWS)])
    pltpu.sync_copy(d_hbm.at[pl.ds(wid * _RPD, _RPD)], dv)
    pltpu.sync_copy(ones_hbm, ones_v)
    plsc.subcore_barrier()

    @pl.loop(0, _RPD)
    def _(r):
        pltpu.sync_copy(ones_v, acc.at[dv.at[r]], add=True)

    plsc.subcore_barrier()
    pltpu.sync_copy(
        acc.at[pl.ds(sid * _ZROWS, _ZROWS)],
        out_hbm.at[cid].at[pl.ds(sid * _ZROWS, _ZROWS)],
    )


_NBUF = 4


@functools.partial(
    pl.kernel,
    out_type=jax.ShapeDtypeStruct((_NC, _NPAD, _DH), jnp.float32),
    mesh=_mesh,
    compiler_params=_sc_params,
    scratch_types=[
        pltpu.VMEM((_RPS, _LW), jnp.int32),
        pltpu.VMEM((_RPS, _LW), jnp.int32),
    ]
    + [pltpu.VMEM((_LW, _DH), jnp.float32)] * _NBUF
    + [pltpu.VMEM_SHARED((_NPAD, _DH), jnp.float32)]
    + [pltpu.SemaphoreType.DMA] * (2 * _NBUF),
)
def _sc_scatter(z_hbm, s_hbm, d_hbm, zeros_hbm, out_hbm,
                sv, dv, rb0, rb1, rb2, rb3, acc,
                gs0, gs1, gs2, gs3, ss0, ss1, ss2, ss3):
    rbs = (rb0, rb1, rb2, rb3)
    gsems = (gs0, gs1, gs2, gs3)
    ssems = (ss0, ss1, ss2, ss3)
    cid = lax.axis_index("c")
    sid = lax.axis_index("s")
    zc = z_hbm.at[cid]
    c0 = pltpu.async_copy(zeros_hbm, acc.at[pl.ds(sid * _ZROWS, _ZROWS)], gs0)
    c1 = pltpu.async_copy(s_hbm.at[pl.ds(sid * _RPS, _RPS)], sv, gs1)
    c2 = pltpu.async_copy(d_hbm.at[pl.ds(sid * _RPS, _RPS)], dv, gs2)
    c0.wait()
    c1.wait()
    c2.wait()
    plsc.subcore_barrier()

    pltpu.async_copy(zc.at[sv.at[0]], rb0, gs0)
    pltpu.async_copy(zc.at[sv.at[1]], rb1, gs1)

    @pl.loop(0, _RPS, step=_NBUF)
    def _(r):
        for b in range(_NBUF):
            row = r + b
            b2 = (b + 2) % _NBUF
            pltpu.make_async_copy(zc.at[sv.at[row]], rbs[b], gsems[b]).wait()
            pltpu.async_copy(rbs[b], acc.at[dv.at[row]], ssems[b], add=True)

            @pl.when(row >= 2)
            def _():
                pltpu.make_async_copy(
                    rbs[b2], acc.at[dv.at[row - 2]], ssems[b2]
                ).wait()

            @pl.when(row + 2 < _RPS)
            def _():
                pltpu.async_copy(zc.at[sv.at[row + 2]], rbs[b2], gsems[b2])

    for row in (_RPS - 2, _RPS - 1):
        b = row % _NBUF
        pltpu.make_async_copy(rbs[b], acc.at[dv.at[row]], ssems[b]).wait()
    plsc.subcore_barrier()
    pltpu.sync_copy(
        acc.at[pl.ds(sid * _ZROWS, _ZROWS)],
        out_hbm.at[cid].at[pl.ds(sid * _ZROWS, _ZROWS)],
    )


_BR = 1000


def _dot(a, b):
    return lax.dot_general(
        a, b, (((1,), (0,)), ((), ())),
        precision=lax.Precision.HIGHEST,
        preferred_element_type=jnp.float32,
    )


def _split_store(z2_ref, z):
    z2_ref[0] = z[:, :_DH]
    z2_ref[1] = z[:, _DH:]


def _tc_lin_body(x_ref, w_ref, g_ref):
    g_ref[...] = _dot(x_ref[...], w_ref[...])


def _tc_lin(x, w0):
    return pl.pallas_call(
        _tc_lin_body,
        grid=(_N // _BR,),
        in_specs=[
            pl.BlockSpec((_BR, _D), lambda i: (i, 0)),
            pl.BlockSpec((_D, _D), lambda i: (0, 0)),
        ],
        out_specs=pl.BlockSpec((_BR, _D), lambda i: (i, 0)),
        out_shape=jax.ShapeDtypeStruct((_N, _D), jnp.float32),
    )(x, w0)


def _tc_scale_body(g_ref, c_ref, z2_ref, dinv_ref):
    deg = c_ref[0, :, 0:1] + c_ref[1, :, 0:1] + 1.0
    dinv = lax.rsqrt(deg)
    _split_store(z2_ref, g_ref[...] * dinv)
    dinv_ref[...] = dinv


def _tc_scale(g, cnt):
    return pl.pallas_call(
        _tc_scale_body,
        grid=(_N // _BR,),
        in_specs=[
            pl.BlockSpec((_BR, _D), lambda i: (i, 0)),
            pl.BlockSpec((_NC, _BR, 16), lambda i: (0, i, 0)),
        ],
        out_specs=[
            pl.BlockSpec((_NC, _BR, _DH), lambda i: (0, i, 0)),
            pl.BlockSpec((_BR, 1), lambda i: (i, 0)),
        ],
        out_shape=[
            jax.ShapeDtypeStruct((_NC, _N, _DH), jnp.float32),
            jax.ShapeDtypeStruct((_N, 1), jnp.float32),
        ],
    )(g, cnt)


def _relu_cat(m_ref, z_ref, dinv, b_ref):
    t = jnp.concatenate(
        [m_ref[0] + z_ref[0], m_ref[1] + z_ref[1]], axis=1
    ) * dinv + b_ref[...]
    return jnp.maximum(t, 0.0)


def _tc_mid_body(m_ref, z_ref, dinv_ref, b_ref, w_ref, o_ref):
    dinv = dinv_ref[...]
    h = _relu_cat(m_ref, z_ref, dinv, b_ref)
    _split_store(o_ref, _dot(h, w_ref[...]) * dinv)


def _tc_mid(m, z2, dinv, b, w):
    return pl.pallas_call(
        _tc_mid_body,
        grid=(_N // _BR,),
        in_specs=[
            pl.BlockSpec((_NC, _BR, _DH), lambda i: (0, i, 0)),
            pl.BlockSpec((_NC, _BR, _DH), lambda i: (0, i, 0)),
            pl.BlockSpec((_BR, 1), lambda i: (i, 0)),
            pl.BlockSpec((1, _D), lambda i: (0, 0)),
            pl.BlockSpec((_D, _D), lambda i: (0, 0)),
        ],
        out_specs=pl.BlockSpec((_NC, _BR, _DH), lambda i: (0, i, 0)),
        out_shape=jax.ShapeDtypeStruct((_NC, _N, _DH), jnp.float32),
    )(m, z2, dinv, b, w)


def _tc_last_body(m_ref, z_ref, dinv_ref, b_ref, w_ref, bc_ref, o_ref):
    h = _relu_cat(m_ref, z_ref, dinv_ref[...], b_ref)
    o_ref[...] = _dot(h, w_ref[...]) + bc_ref[...]


def _tc_last(m, z2, dinv, b, wc, bc):
    dout = wc.shape[1]
    return pl.pallas_call(
        _tc_last_body,
        grid=(_N // _BR,),
        in_specs=[
            pl.BlockSpec((_NC, _BR, _DH), lambda i: (0, i, 0)),
            pl.BlockSpec((_NC, _BR, _DH), lambda i: (0, i, 0)),
            pl.BlockSpec((_BR, 1), lambda i: (i, 0)),
            pl.BlockSpec((1, _D), lambda i: (0, 0)),
            pl.BlockSpec((_D, dout), lambda i: (0, 0)),
            pl.BlockSpec((1, dout), lambda i: (0, 0)),
        ],
        out_specs=pl.BlockSpec((_BR, dout), lambda i: (i, 0)),
        out_shape=jax.ShapeDtypeStruct((_N, dout), jnp.float32),
    )(m, z2, dinv, b, wc, bc.reshape(1, dout))


def kernel(x, edge_index, W0, b0, W1, b1, W2, b2, Wc, bc):
    s = edge_index[0].astype(jnp.int32)
    d = edge_index[1].astype(jnp.int32)
    npad = _EPAD - _E
    pad_src = jnp.arange(npad, dtype=jnp.int32) % _N
    pad_dst = _N + (jnp.arange(npad, dtype=jnp.int32) % _NW)
    s_rows = jnp.concatenate([s, pad_src]).reshape(_ROWS, _LW)
    d_rows = jnp.concatenate([d, pad_dst]).reshape(_ROWS, _LW)

    ones16 = jnp.ones((_LW, 16), jnp.float32)
    zeros16 = jnp.zeros((_ZROWS, 16), jnp.float32)
    zeros64 = jnp.zeros((_ZROWS, _DH), jnp.float32)

    def _sc(z2):
        return _sc_scatter(z2, s_rows, d_rows, zeros64)

    cnt = _sc_degree(d_rows, ones16, zeros16)
    g0 = _tc_lin(x, W0)
    z0, dinv = _tc_scale(g0, cnt)

    z1 = _tc_mid(_sc(z0), z0, dinv, b0.reshape(1, _D), W1)
    z2 = _tc_mid(_sc(z1), z1, dinv, b1.reshape(1, _D), W2)
    return _tc_last(_sc(z2), z2, dinv, b2.reshape(1, _D), Wc, bc)
